# baseline (speedup 1.0000x reference)
import functools

import jax
import jax.numpy as jnp
from jax import lax
from jax.experimental import pallas as pl
from jax.experimental.pallas import tpu as pltpu
from jax.experimental.pallas import tpu_sc as plsc

NT = 16
NJ = 2048
B = 16384
H = 64

_SC = plsc.get_sparse_core_info()
_NC = _SC.num_cores
_NS = _SC.num_subcores
_L = _SC.num_lanes
NW = _NC * _NS
B_PER_W = B // NW
CHUNK = 128
NCHUNK = B_PER_W // CHUNK


def _gather_body(flat_hbm, i0_hbm, i1_hbm, out_hbm, i0_v, i1_v, fidx_v, x_v, sem):
    wid = lax.axis_index("s") * _NC + lax.axis_index("c")
    base = wid * B_PER_W
    pltpu.sync_copy(i0_hbm.at[pl.ds(base, B_PER_W)], i0_v)
    pltpu.sync_copy(i1_hbm.at[pl.ds(base, B_PER_W)], i1_v)
    for j in range(NCHUNK):
        for k in range(CHUNK // _L):
            off = j * CHUNK + k * _L
            a = i0_v[pl.ds(off, _L)]
            b = i1_v[pl.ds(off, _L)]
            fidx_v[j, pl.ds(k * _L, _L)] = a * NJ + b
    copies = [
        pltpu.async_copy(flat_hbm.at[fidx_v.at[j]],
                         x_v.at[pl.ds(j * CHUNK, CHUNK)], sem)
        for j in range(NCHUNK)
    ]
    for c in copies:
        c.wait()
    pltpu.sync_copy(x_v, out_hbm.at[pl.ds(base, B_PER_W)])


_gather = functools.partial(
    pl.kernel,
    mesh=plsc.VectorSubcoreMesh(core_axis_name="c", subcore_axis_name="s"),
    out_type=jax.ShapeDtypeStruct((B,), jnp.float32),
    scratch_types=[
        pltpu.VMEM((B_PER_W,), jnp.int32),
        pltpu.VMEM((B_PER_W,), jnp.int32),
        pltpu.VMEM((NCHUNK, CHUNK), jnp.int32),
        pltpu.VMEM((B_PER_W,), jnp.float32),
        pltpu.SemaphoreType.DMA,
    ],
)(_gather_body)


BLK = 512
CPR = B // BLK


def _loss_body(x_ref, w1_ref, b1_ref, w2t_ref, b2_ref, out_ref, X_ref):
    w1 = w1_ref[...]
    b1 = b1_ref[...]
    w2t = w2t_ref[...]
    b2 = b2_ref[...]

    x = x_ref[...]
    for t in range(NT - 1):
        x = x + 0.01 * jnp.tanh(x)
        X_ref[pl.ds(t, 1), :] = x

    b1b = b1 * jnp.ones((1, BLK), jnp.float32)
    dn = (((1,), (0,)), ((), ()))

    def tile(n, accv):
        r = n // CPR
        c = n % CPR
        xb = X_ref[pl.ds(r, 1), pl.ds(c * BLK, BLK)]
        hp = lax.dot_general(w1, xb, dn,
                             preferred_element_type=jnp.float32)
        h = jnp.tanh(hp + b1b)
        x2 = lax.dot_general(w2t, h, dn,
                             preferred_element_type=jnp.float32)
        d = xb - (x2 + b2)
        return accv + d * d

    accv = lax.fori_loop(0, (NT - 1) * CPR, tile,
                         jnp.zeros((1, BLK), jnp.float32))
    out_ref[...] = jnp.reshape(jnp.sum(accv), (1, 1)) / jnp.float32((NT - 1) * B)


_loss = pl.pallas_call(
    _loss_body,
    out_shape=jax.ShapeDtypeStruct((1, 1), jnp.float32),
    scratch_shapes=[pltpu.VMEM((NT - 1, B), jnp.float32)],
)


def kernel(xi0, W1, b1, W2, b2, index):
    flat = xi0.reshape(-1)
    idx = index.astype(jnp.int32)
    x0 = _gather(flat, idx[:, 0], idx[:, 1])
    out = _loss(x0.reshape(1, B), W1.reshape(H, 1), b1.reshape(H, 1),
                W2.reshape(1, H), b2.reshape(1, 1))
    return out[0, 0]


# speedup vs baseline: 2.0802x; 2.0802x over previous
import functools

import jax
import jax.numpy as jnp
from jax import lax
from jax.experimental import pallas as pl
from jax.experimental.pallas import tpu as pltpu
from jax.experimental.pallas import tpu_sc as plsc

NT = 16
NJ = 2048
B = 16384
H = 64

_SC = plsc.get_sparse_core_info()
_NC = _SC.num_cores
_NS = _SC.num_subcores
_L = _SC.num_lanes
NW = _NC * _NS
B_PER_W = B // NW
CHUNK = 128
NCHUNK = B_PER_W // CHUNK


def _gather_body(flat_hbm, i0_hbm, i1_hbm, out_hbm, i0_v, i1_v, fidx_v, x_v, sem):
    wid = lax.axis_index("s") * _NC + lax.axis_index("c")
    base = wid * B_PER_W
    pltpu.sync_copy(i0_hbm.at[pl.ds(base, B_PER_W)], i0_v)
    pltpu.sync_copy(i1_hbm.at[pl.ds(base, B_PER_W)], i1_v)
    for j in range(NCHUNK):
        for k in range(CHUNK // _L):
            off = j * CHUNK + k * _L
            a = i0_v[pl.ds(off, _L)]
            b = i1_v[pl.ds(off, _L)]
            fidx_v[j, pl.ds(k * _L, _L)] = a * NJ + b
    copies = [
        pltpu.async_copy(flat_hbm.at[fidx_v.at[j]],
                         x_v.at[pl.ds(j * CHUNK, CHUNK)], sem)
        for j in range(NCHUNK)
    ]
    for c in copies:
        c.wait()
    pltpu.sync_copy(x_v, out_hbm.at[pl.ds(base, B_PER_W)])


_gather = functools.partial(
    pl.kernel,
    mesh=plsc.VectorSubcoreMesh(core_axis_name="c", subcore_axis_name="s"),
    out_type=jax.ShapeDtypeStruct((B,), jnp.float32),
    scratch_types=[
        pltpu.VMEM((B_PER_W,), jnp.int32),
        pltpu.VMEM((B_PER_W,), jnp.int32),
        pltpu.VMEM((NCHUNK, CHUNK), jnp.int32),
        pltpu.VMEM((B_PER_W,), jnp.float32),
        pltpu.SemaphoreType.DMA,
    ],
)(_gather_body)


CH = 2048
SUB = B // CH
NCH = (NT - 1) * SUB


def _loss_body(x_ref, w1_ref, b1_ref, w2t_ref, b2_ref, out_ref, X_ref, X2_ref):
    w1aug = jnp.concatenate([w1_ref[...], b1_ref[...]], axis=1)
    w2t = w2t_ref[...]
    ones_row = jnp.ones((1, CH), jnp.float32)
    dn = (((1,), (0,)), ((), ()))

    x = x_ref[...]
    for t in range(NT - 1):
        x = x + 0.01 * jnp.tanh(x)
        X_ref[pl.ds(t, 1), :, :] = x.reshape(1, SUB, CH)

    def chunk(n, carry):
        t = n // SUB
        s = n % SUB
        xb = X_ref[pl.ds(t, 1), pl.ds(s, 1), :].reshape(1, CH)
        xaug = jnp.concatenate([xb, ones_row], axis=0)
        h = jnp.tanh(lax.dot_general(w1aug, xaug, dn,
                                     preferred_element_type=jnp.float32))
        x2 = lax.dot_general(w2t, h, dn,
                             preferred_element_type=jnp.float32)
        X2_ref[pl.ds(t, 1), pl.ds(s, 1), :] = x2.reshape(1, 1, CH)
        return carry

    lax.fori_loop(0, NCH, chunk, 0)

    b2v = b2_ref[...].reshape(1, 1, 1)
    d = X_ref[...] - X2_ref[...] - b2v
    out_ref[...] = jnp.reshape(jnp.sum(d * d), (1, 1)) / jnp.float32((NT - 1) * B)


_loss = pl.pallas_call(
    _loss_body,
    out_shape=jax.ShapeDtypeStruct((1, 1), jnp.float32),
    scratch_shapes=[
        pltpu.VMEM((NT - 1, SUB, CH), jnp.float32),
        pltpu.VMEM((NT - 1, SUB, CH), jnp.float32),
    ],
)


def kernel(xi0, W1, b1, W2, b2, index):
    flat = xi0.reshape(-1)
    idx = index.astype(jnp.int32)
    x0 = _gather(flat, idx[:, 0], idx[:, 1])
    out = _loss(x0.reshape(SUB, CH), W1.reshape(H, 1), b1.reshape(H, 1),
                W2.reshape(1, H), b2.reshape(1, 1))
    return out[0, 0]


# speedup vs baseline: 3.2326x; 1.5540x over previous
import functools

import jax
import jax.numpy as jnp
from jax import lax
from jax.experimental import pallas as pl
from jax.experimental.pallas import tpu as pltpu
from jax.experimental.pallas import tpu_sc as plsc

NT = 16
NJ = 2048
B = 16384
H = 64

_SC = plsc.get_sparse_core_info()
_NC = _SC.num_cores
_NS = _SC.num_subcores
_L = _SC.num_lanes
NW = _NC * _NS
B_PER_W = B // NW
CHUNK = 128
NCHUNK = B_PER_W // CHUNK


def _gather_body(flat_hbm, i0_hbm, i1_hbm, out_hbm, i0_v, i1_v, fidx_v, x_v, sem):
    wid = lax.axis_index("s") * _NC + lax.axis_index("c")
    base = wid * B_PER_W
    pltpu.sync_copy(i0_hbm.at[pl.ds(base, B_PER_W)], i0_v)
    pltpu.sync_copy(i1_hbm.at[pl.ds(base, B_PER_W)], i1_v)
    for j in range(NCHUNK):
        for k in range(CHUNK // _L):
            off = j * CHUNK + k * _L
            a = i0_v[pl.ds(off, _L)]
            b = i1_v[pl.ds(off, _L)]
            fidx_v[j, pl.ds(k * _L, _L)] = a * NJ + b
    copies = [
        pltpu.async_copy(flat_hbm.at[fidx_v.at[j]],
                         x_v.at[pl.ds(j * CHUNK, CHUNK)], sem)
        for j in range(NCHUNK)
    ]
    for c in copies:
        c.wait()
    pltpu.sync_copy(x_v, out_hbm.at[pl.ds(base, B_PER_W)])


_gather = functools.partial(
    pl.kernel,
    mesh=plsc.VectorSubcoreMesh(core_axis_name="c", subcore_axis_name="s"),
    out_type=jax.ShapeDtypeStruct((B,), jnp.float32),
    scratch_types=[
        pltpu.VMEM((B_PER_W,), jnp.int32),
        pltpu.VMEM((B_PER_W,), jnp.int32),
        pltpu.VMEM((NCHUNK, CHUNK), jnp.int32),
        pltpu.VMEM((B_PER_W,), jnp.float32),
        pltpu.SemaphoreType.DMA,
    ],
)(_gather_body)


def _loss_body(x_ref, w1_ref, b1_ref, w2t_ref, b2_ref, out_ref):
    w1aug = jnp.concatenate([w1_ref[...], b1_ref[...]], axis=1)
    w2t = w2t_ref[...]
    b2 = b2_ref[...]
    ones_row = jnp.ones((1, B), jnp.float32)
    dn = (((1,), (0,)), ((), ()))

    x = x_ref[...]
    accv = jnp.zeros((1, B), jnp.float32)
    for t in range(NT - 1):
        x = x + 0.01 * jnp.tanh(x)
        xaug = jnp.concatenate([x, ones_row], axis=0)
        h = jnp.tanh(lax.dot_general(w1aug, xaug, dn,
                                     preferred_element_type=jnp.float32))
        x2 = lax.dot_general(w2t, h, dn,
                             preferred_element_type=jnp.float32)
        d = x - x2 - b2
        accv = accv + d * d

    out_ref[...] = jnp.reshape(jnp.sum(accv), (1, 1)) / jnp.float32((NT - 1) * B)


_loss = pl.pallas_call(
    _loss_body,
    out_shape=jax.ShapeDtypeStruct((1, 1), jnp.float32),
)


def kernel(xi0, W1, b1, W2, b2, index):
    flat = xi0.reshape(-1)
    idx = index.astype(jnp.int32)
    x0 = _gather(flat, idx[:, 0], idx[:, 1])
    out = _loss(x0.reshape(1, B), W1.reshape(H, 1), b1.reshape(H, 1),
                W2.reshape(1, H), b2.reshape(1, 1))
    return out[0, 0]
